# Initial kernel scaffold; baseline (speedup 1.0000x reference)
import jax
import jax.numpy as jnp
from jax.experimental import pallas as pl


def kernel(x, pe):
    raise NotImplementedError("write your pallas kernel here")


# speedup vs baseline: 2.8503x; 2.8503x over previous
import jax
import jax.numpy as jnp
from jax.experimental import pallas as pl


_S_BLK = 512


def _add_pe_kernel(x_ref, pe_ref, o_ref):
    o_ref[...] = x_ref[...] + pe_ref[...][None, :, :]


def kernel(x, pe):
    bs, seq, d = x.shape
    pe = pe[:seq]
    grid = (seq // _S_BLK, bs)
    return pl.pallas_call(
        _add_pe_kernel,
        grid=grid,
        in_specs=[
            pl.BlockSpec((1, _S_BLK, d), lambda s, b: (b, s, 0)),
            pl.BlockSpec((_S_BLK, d), lambda s, b: (s, 0)),
        ],
        out_specs=pl.BlockSpec((1, _S_BLK, d), lambda s, b: (b, s, 0)),
        out_shape=jax.ShapeDtypeStruct((bs, seq, d), x.dtype),
    )(x, pe)
